# Initial kernel scaffold; baseline (speedup 1.0000x reference)
import jax
import jax.numpy as jnp
from jax.experimental import pallas as pl


def kernel(tokens, byte_table, hash_tables, W_in, b_in, W1, b1, g1, be1, W2, b2, g2, be2, W_out, b_out):
    raise NotImplementedError("write your pallas kernel here")


# speedup vs baseline: 12.4960x; 12.4960x over previous
import functools

import jax
import jax.numpy as jnp
import numpy as np
from jax import lax
from jax.experimental import pallas as pl
from jax.experimental.pallas import tpu as pltpu
from jax.experimental.pallas import tpu_sc as plsc

_HASH_PRIMES = [2654435761, 2246822519, 3266489917, 2028178513, 1220703125, 1610612741, 805306457, 402653189, 3674653429, 2860486313, 1073676287, 2971215073, 1500450271, 3267000013, 2654435789, 4049292737, 2246822531, 3266489927, 2028178519, 1220703133, 1610612743, 805306459, 402653191, 3674653433, 2654435771, 2246822527, 3266489933, 2028178529, 1220703137, 1610612747, 805306463, 402653197, 3674653441, 2860486319, 1073676293, 2971215077, 1500450281, 3267000017, 2654435801, 4049292743, 2246822537, 3266489939, 2028178531, 1220703143, 1610612753, 805306467, 402653201, 3674653447]

_BATCH = 1024
_SEQ = 20
_N = _BATCH * _SEQ
_NUM_TABLES = 16
_BUCKETS = 100000
_BYTE_DIM = 128
_EMB = 16
_NF = _NUM_TABLES * _EMB
_IN_DIM = _BYTE_DIM + _NF
_HIDDEN = 512
_VOCAB = 1024


def _default_patterns(num_tables):
    patterns = []
    for offset in range(1, min(num_tables // 4 + 1, 9)):
        patterns.append((offset,))
    pairs = [(1, 2), (2, 3), (3, 4), (1, 3), (2, 4), (1, 4), (1, 5), (2, 5), (3, 5), (1, 6), (2, 6), (1, 7)]
    for p in pairs:
        if len(patterns) >= num_tables:
            break
        patterns.append(p)
    trigrams = [(1, 2, 3), (1, 2, 4), (1, 3, 5), (2, 3, 4)]
    for t in trigrams:
        if len(patterns) >= num_tables:
            break
        patterns.append(t)
    offset = 8
    while len(patterns) < num_tables:
        patterns.append((1, offset))
        offset += 1
    return tuple(patterns[:num_tables])


_PATTERNS = _default_patterns(_NUM_TABLES)
_MAX_OFF = max(max(p) for p in _PATTERNS)
_NB = _N // 128


def _mod_1e5(u):
    q = (u.astype(jnp.float32) * jnp.float32(1e-5)).astype(jnp.int32)
    r = u - q * 100000
    r = jnp.where(r < 0, r + 100000, r)
    r = jnp.where(r < 0, r + 100000, r)
    r = jnp.where(r >= 100000, r - 100000, r)
    r = jnp.where(r >= 100000, r - 100000, r)
    return r


def _hash_body(shift_ref, out_ref):
    shifts = [shift_ref[o] for o in range(_MAX_OFF)]
    c16 = jnp.int32(16)
    for t, pattern in enumerate(_PATTERNS):
        h_lo = jnp.zeros_like(shifts[0])
        h_hi = jnp.zeros_like(shifts[0])
        for k, off in enumerate(pattern):
            p = _HASH_PRIMES[(t * 3 + k) % len(_HASH_PRIMES)]
            p_hi, p_lo = p >> 16, p & 0xFFFF
            tok = shifts[off - 1]
            a = tok * p_hi
            b = tok * p_lo
            c = a + lax.shift_right_logical(b, c16)
            hi = lax.shift_right_logical(c, c16)
            lo = lax.shift_left(c & 0xFFFF, c16) | (b & 0xFFFF)
            h_hi = h_hi ^ hi
            h_lo = h_lo ^ lo
        s = lax.shift_right_logical(h_lo, jnp.int32(31))
        u = h_lo & 0x7FFFFFFF
        v = h_hi * 67296 + s * 83648 + _mod_1e5(u)
        base_e = (_mod_1e5(v) + t * _BUCKETS) * 16
        for e in range(_EMB):
            out_ref[t * _EMB + e] = base_e + e


def _compute_indices(shift_stack):
    return pl.pallas_call(
        _hash_body,
        out_shape=jax.ShapeDtypeStruct((_NF, _NB, 128), jnp.int32),
    )(shift_stack)


_NC = 2
_NS = 16
_NW = _NC * _NS
_C = 128
_SUB = _N // _NW // _C


def _sc_gather_body(tok_hbm, eidx_hbm, byte_hbm, tabs_hbm, byte_out, xt_out,
                    tok_v, eidx_v, byte_v, xt_v, sem):
    c = lax.axis_index("c")
    s = lax.axis_index("s")
    wid = s * jnp.int32(_NC) + c

    def body(j, carry):
        bb = wid * jnp.int32(_SUB) + j
        base = bb * jnp.int32(_C)
        pltpu.sync_copy(tok_hbm.at[pl.ds(base, _C)], tok_v)
        pltpu.sync_copy(eidx_hbm.at[:, bb], eidx_v)
        cp_byte = pltpu.async_copy(byte_hbm.at[tok_v], byte_v, sem)

        def inner(kk, cc):
            k0 = kk * jnp.int32(16)
            for u in range(16):
                k = k0 + jnp.int32(u)
                pltpu.async_copy(tabs_hbm.at[eidx_v.at[k]], xt_v.at[k], sem)
            pltpu.make_async_copy(
                xt_out.at[pl.ds(k0, 16), pl.ds(base, _C)],
                xt_v.at[pl.ds(k0, 16)], sem).wait()
            return cc

        lax.fori_loop(jnp.int32(0), jnp.int32(_NF // 16), inner, jnp.int32(0))
        cp_byte.wait()
        pltpu.sync_copy(byte_v, byte_out.at[pl.ds(base, _C)])
        pltpu.sync_copy(xt_v, xt_out.at[:, pl.ds(base, _C)])
        return carry

    lax.fori_loop(jnp.int32(0), jnp.int32(_SUB), body, jnp.int32(0))


def _sc_gather(tok_flat, eidx, byte_table, tabs_flat):
    mesh = plsc.VectorSubcoreMesh(core_axis_name="c", subcore_axis_name="s")
    f = functools.partial(
        pl.kernel,
        mesh=mesh,
        out_type=(
            jax.ShapeDtypeStruct((_N, _BYTE_DIM), jnp.float32),
            jax.ShapeDtypeStruct((_NF, _N), jnp.float32),
        ),
        scratch_types=[
            pltpu.VMEM((_C,), jnp.int32),
            pltpu.VMEM((_NF, _C), jnp.int32),
            pltpu.VMEM((_C, _BYTE_DIM), jnp.float32),
            pltpu.VMEM((_NF, _C), jnp.float32),
            pltpu.SemaphoreType.DMA,
        ],
    )(_sc_gather_body)
    return f(tok_flat, eidx, byte_table, tabs_flat)


_BN = 1024


def _dense_body(byte_ref, xt_ref, Wb_ref, Wt_ref, bin_ref, W1_ref, b1_ref,
                g1_ref, be1_ref, W2_ref, b2_ref, g2_ref, be2_ref,
                Wout_ref, bout_ref, o_ref):
    x = jnp.dot(byte_ref[...], Wb_ref[...], preferred_element_type=jnp.float32)
    x = x + lax.dot_general(xt_ref[...], Wt_ref[...],
                            (((0,), (0,)), ((), ())),
                            preferred_element_type=jnp.float32)
    x = x + bin_ref[...]
    for W_ref, b_ref, g_ref, be_ref in ((W1_ref, b1_ref, g1_ref, be1_ref),
                                        (W2_ref, b2_ref, g2_ref, be2_ref)):
        h = jnp.maximum(jnp.dot(x, W_ref[...], preferred_element_type=jnp.float32) + b_ref[...], 0.0)
        r = h + x
        mu = jnp.mean(r, axis=-1, keepdims=True)
        var = jnp.mean((r - mu) ** 2, axis=-1, keepdims=True)
        x = (r - mu) / jnp.sqrt(var + 1e-5) * g_ref[...] + be_ref[...]
    o_ref[...] = jnp.dot(x, Wout_ref[...], preferred_element_type=jnp.float32) + bout_ref[...]


def _dense(byte_e, xt, W_in, b_in, W1, b1, g1, be1, W2, b2, g2, be2, W_out, b_out):
    grid = (_N // _BN,)
    _z = np.int32(0)
    full = lambda shape: pl.BlockSpec(shape, lambda i: (_z, _z))
    return pl.pallas_call(
        _dense_body,
        grid=grid,
        in_specs=[
            pl.BlockSpec((_BN, _BYTE_DIM), lambda i: (i, _z)),
            pl.BlockSpec((_NF, _BN), lambda i: (_z, i)),
            full((_BYTE_DIM, _HIDDEN)), full((_NF, _HIDDEN)), full((1, _HIDDEN)),
            full((_HIDDEN, _HIDDEN)), full((1, _HIDDEN)), full((1, _HIDDEN)), full((1, _HIDDEN)),
            full((_HIDDEN, _HIDDEN)), full((1, _HIDDEN)), full((1, _HIDDEN)), full((1, _HIDDEN)),
            full((_HIDDEN, _VOCAB)), full((1, _VOCAB)),
        ],
        out_specs=pl.BlockSpec((_BN, _VOCAB), lambda i: (i, _z)),
        out_shape=jax.ShapeDtypeStruct((_N, _VOCAB), jnp.float32),
    )(byte_e, xt, W_in[:_BYTE_DIM], W_in[_BYTE_DIM:], b_in.reshape(1, -1),
      W1, b1.reshape(1, -1), g1.reshape(1, -1), be1.reshape(1, -1),
      W2, b2.reshape(1, -1), g2.reshape(1, -1), be2.reshape(1, -1),
      W_out, b_out.reshape(1, -1))


def kernel(tokens, byte_table, hash_tables, W_in, b_in, W1, b1, g1, be1,
           W2, b2, g2, be2, W_out, b_out):
    out_dtype = jnp.result_type(byte_table.dtype, W_in.dtype, W_out.dtype)
    f32 = jnp.float32
    byte_table = byte_table.astype(f32)
    hash_tables = hash_tables.astype(f32)
    W_in, b_in, W1, b1, g1, be1 = (a.astype(f32) for a in (W_in, b_in, W1, b1, g1, be1))
    W2, b2, g2, be2, W_out, b_out = (a.astype(f32) for a in (W2, b2, g2, be2, W_out, b_out))
    tok32 = tokens.astype(jnp.int32)
    shifts = [jnp.pad(tok32[:, :-o], ((0, 0), (o, 0))) for o in range(1, _MAX_OFF + 1)]
    shift_stack = jnp.stack(shifts).reshape(_MAX_OFF, _NB, 128)
    eidx = _compute_indices(shift_stack)
    tok_flat = tok32.reshape(_N)
    tabs_flat = hash_tables.reshape(_NUM_TABLES * _BUCKETS * _EMB)
    byte_e, xt = _sc_gather(tok_flat, eidx, byte_table, tabs_flat)
    out = _dense(byte_e, xt, W_in, b_in, W1, b1, g1, be1, W2, b2, g2, be2,
                 W_out, b_out)
    return out.reshape(_BATCH, _SEQ, _VOCAB).astype(out_dtype)


# speedup vs baseline: 16.3362x; 1.3073x over previous
import functools

import jax
import jax.numpy as jnp
import numpy as np
from jax import lax
from jax.experimental import pallas as pl
from jax.experimental.pallas import tpu as pltpu
from jax.experimental.pallas import tpu_sc as plsc

_HASH_PRIMES = [2654435761, 2246822519, 3266489917, 2028178513, 1220703125, 1610612741, 805306457, 402653189, 3674653429, 2860486313, 1073676287, 2971215073, 1500450271, 3267000013, 2654435789, 4049292737, 2246822531, 3266489927, 2028178519, 1220703133, 1610612743, 805306459, 402653191, 3674653433, 2654435771, 2246822527, 3266489933, 2028178529, 1220703137, 1610612747, 805306463, 402653197, 3674653441, 2860486319, 1073676293, 2971215077, 1500450281, 3267000017, 2654435801, 4049292743, 2246822537, 3266489939, 2028178531, 1220703143, 1610612753, 805306467, 402653201, 3674653447]

_BATCH = 1024
_SEQ = 20
_N = _BATCH * _SEQ
_NUM_TABLES = 16
_BUCKETS = 100000
_BYTE_DIM = 128
_EMB = 16
_NF = _NUM_TABLES * _EMB
_IN_DIM = _BYTE_DIM + _NF
_HIDDEN = 512
_VOCAB = 1024


def _default_patterns(num_tables):
    patterns = []
    for offset in range(1, min(num_tables // 4 + 1, 9)):
        patterns.append((offset,))
    pairs = [(1, 2), (2, 3), (3, 4), (1, 3), (2, 4), (1, 4), (1, 5), (2, 5), (3, 5), (1, 6), (2, 6), (1, 7)]
    for p in pairs:
        if len(patterns) >= num_tables:
            break
        patterns.append(p)
    trigrams = [(1, 2, 3), (1, 2, 4), (1, 3, 5), (2, 3, 4)]
    for t in trigrams:
        if len(patterns) >= num_tables:
            break
        patterns.append(t)
    offset = 8
    while len(patterns) < num_tables:
        patterns.append((1, offset))
        offset += 1
    return tuple(patterns[:num_tables])


_PATTERNS = _default_patterns(_NUM_TABLES)
_MAX_OFF = max(max(p) for p in _PATTERNS)
_NB = _N // 128


def _mod_1e5(u):
    q = (u.astype(jnp.float32) * jnp.float32(1e-5)).astype(jnp.int32)
    r = u - q * 100000
    r = jnp.where(r < 0, r + 100000, r)
    r = jnp.where(r < 0, r + 100000, r)
    r = jnp.where(r >= 100000, r - 100000, r)
    r = jnp.where(r >= 100000, r - 100000, r)
    return r


def _hash_body(shift_ref, out_ref):
    shifts = [shift_ref[o] for o in range(_MAX_OFF)]
    c16 = jnp.int32(16)
    for t, pattern in enumerate(_PATTERNS):
        h_lo = jnp.zeros_like(shifts[0])
        h_hi = jnp.zeros_like(shifts[0])
        for k, off in enumerate(pattern):
            p = _HASH_PRIMES[(t * 3 + k) % len(_HASH_PRIMES)]
            p_hi, p_lo = p >> 16, p & 0xFFFF
            tok = shifts[off - 1]
            a = tok * p_hi
            b = tok * p_lo
            c = a + lax.shift_right_logical(b, c16)
            hi = lax.shift_right_logical(c, c16)
            lo = lax.shift_left(c & 0xFFFF, c16) | (b & 0xFFFF)
            h_hi = h_hi ^ hi
            h_lo = h_lo ^ lo
        s = lax.shift_right_logical(h_lo, jnp.int32(31))
        u = h_lo & 0x7FFFFFFF
        v = h_hi * 67296 + s * 83648 + _mod_1e5(u)
        idx = _mod_1e5(v)
        for e in range(_EMB):
            out_ref[t * _EMB + e] = idx + (t * _EMB + e) * _BUCKETS


def _compute_indices(shift_stack):
    return pl.pallas_call(
        _hash_body,
        out_shape=jax.ShapeDtypeStruct((_NF, _NB, 128), jnp.int32),
    )(shift_stack)


_NC = 2
_NS = 16
_NW = _NC * _NS
_C = 128
_SUB = _N // _NW // _C


def _sc_gather_body(tok_hbm, eidx_hbm, byte_hbm, tabs_hbm, byte_out, xt_out,
                    tok_v, eidx_v, byte_v, xt_v, sem):
    c = lax.axis_index("c")
    s = lax.axis_index("s")
    wid = s * jnp.int32(_NC) + c

    def body(j, carry):
        bb = wid * jnp.int32(_SUB) + j
        base = bb * jnp.int32(_C)
        pltpu.sync_copy(tok_hbm.at[pl.ds(base, _C)], tok_v)
        pltpu.sync_copy(eidx_hbm.at[:, bb], eidx_v)
        cp_byte = pltpu.async_copy(byte_hbm.at[tok_v], byte_v, sem)

        def inner(kk, cc):
            k0 = kk * jnp.int32(16)
            for u in range(16):
                k = k0 + jnp.int32(u)
                pltpu.async_copy(tabs_hbm.at[eidx_v.at[k]], xt_v.at[k], sem)
            pltpu.make_async_copy(
                xt_out.at[pl.ds(k0, 16), pl.ds(base, _C)],
                xt_v.at[pl.ds(k0, 16)], sem).wait()
            return cc

        lax.fori_loop(jnp.int32(0), jnp.int32(_NF // 16), inner, jnp.int32(0))
        cp_byte.wait()
        pltpu.sync_copy(byte_v, byte_out.at[pl.ds(base, _C)])
        pltpu.sync_copy(xt_v, xt_out.at[:, pl.ds(base, _C)])
        return carry

    lax.fori_loop(jnp.int32(0), jnp.int32(_SUB), body, jnp.int32(0))


def _sc_gather(tok_flat, eidx, byte_table, tabs_flat):
    mesh = plsc.VectorSubcoreMesh(core_axis_name="c", subcore_axis_name="s")
    f = functools.partial(
        pl.kernel,
        mesh=mesh,
        out_type=(
            jax.ShapeDtypeStruct((_N, _BYTE_DIM), jnp.float32),
            jax.ShapeDtypeStruct((_NF, _N), jnp.float32),
        ),
        scratch_types=[
            pltpu.VMEM((_C,), jnp.int32),
            pltpu.VMEM((_NF, _C), jnp.int32),
            pltpu.VMEM((_C, _BYTE_DIM), jnp.float32),
            pltpu.VMEM((_NF, _C), jnp.float32),
            pltpu.SemaphoreType.DMA,
        ],
    )(_sc_gather_body)
    return f(tok_flat, eidx, byte_table, tabs_flat)


_BN = 1024


def _dense_body(byte_ref, xt_ref, Wb_ref, Wt_ref, bin_ref, W1_ref, b1_ref,
                g1_ref, be1_ref, W2_ref, b2_ref, g2_ref, be2_ref,
                Wout_ref, bout_ref, o_ref):
    x = jnp.dot(byte_ref[...], Wb_ref[...], preferred_element_type=jnp.float32)
    x = x + lax.dot_general(xt_ref[...], Wt_ref[...],
                            (((0,), (0,)), ((), ())),
                            preferred_element_type=jnp.float32)
    x = x + bin_ref[...]
    for W_ref, b_ref, g_ref, be_ref in ((W1_ref, b1_ref, g1_ref, be1_ref),
                                        (W2_ref, b2_ref, g2_ref, be2_ref)):
        h = jnp.maximum(jnp.dot(x, W_ref[...], preferred_element_type=jnp.float32) + b_ref[...], 0.0)
        r = h + x
        mu = jnp.mean(r, axis=-1, keepdims=True)
        var = jnp.mean((r - mu) ** 2, axis=-1, keepdims=True)
        x = (r - mu) / jnp.sqrt(var + 1e-5) * g_ref[...] + be_ref[...]
    o_ref[...] = jnp.dot(x, Wout_ref[...], preferred_element_type=jnp.float32) + bout_ref[...]


def _dense(byte_e, xt, W_in, b_in, W1, b1, g1, be1, W2, b2, g2, be2, W_out, b_out):
    grid = (_N // _BN,)
    _z = np.int32(0)
    full = lambda shape: pl.BlockSpec(shape, lambda i: (_z, _z))
    return pl.pallas_call(
        _dense_body,
        grid=grid,
        in_specs=[
            pl.BlockSpec((_BN, _BYTE_DIM), lambda i: (i, _z)),
            pl.BlockSpec((_NF, _BN), lambda i: (_z, i)),
            full((_BYTE_DIM, _HIDDEN)), full((_NF, _HIDDEN)), full((1, _HIDDEN)),
            full((_HIDDEN, _HIDDEN)), full((1, _HIDDEN)), full((1, _HIDDEN)), full((1, _HIDDEN)),
            full((_HIDDEN, _HIDDEN)), full((1, _HIDDEN)), full((1, _HIDDEN)), full((1, _HIDDEN)),
            full((_HIDDEN, _VOCAB)), full((1, _VOCAB)),
        ],
        out_specs=pl.BlockSpec((_BN, _VOCAB), lambda i: (i, _z)),
        out_shape=jax.ShapeDtypeStruct((_N, _VOCAB), jnp.float32),
    )(byte_e, xt, W_in[:_BYTE_DIM], W_in[_BYTE_DIM:], b_in.reshape(1, -1),
      W1, b1.reshape(1, -1), g1.reshape(1, -1), be1.reshape(1, -1),
      W2, b2.reshape(1, -1), g2.reshape(1, -1), be2.reshape(1, -1),
      W_out, b_out.reshape(1, -1))


def kernel(tokens, byte_table, hash_tables, W_in, b_in, W1, b1, g1, be1,
           W2, b2, g2, be2, W_out, b_out):
    out_dtype = jnp.result_type(byte_table.dtype, W_in.dtype, W_out.dtype)
    f32 = jnp.float32
    byte_table = byte_table.astype(f32)
    hash_tables = hash_tables.astype(f32)
    W_in, b_in, W1, b1, g1, be1 = (a.astype(f32) for a in (W_in, b_in, W1, b1, g1, be1))
    W2, b2, g2, be2, W_out, b_out = (a.astype(f32) for a in (W2, b2, g2, be2, W_out, b_out))
    tok32 = tokens.astype(jnp.int32)
    shifts = [jnp.pad(tok32[:, :-o], ((0, 0), (o, 0))) for o in range(1, _MAX_OFF + 1)]
    shift_stack = jnp.stack(shifts).reshape(_MAX_OFF, _NB, 128)
    eidx = _compute_indices(shift_stack)
    tok_flat = tok32.reshape(_N)
    tabs_flat = jnp.transpose(hash_tables, (0, 2, 1)).reshape(
        _NUM_TABLES * _EMB * _BUCKETS)
    byte_e, xt = _sc_gather(tok_flat, eidx, byte_table, tabs_flat)
    out = _dense(byte_e, xt, W_in, b_in, W1, b1, g1, be1, W2, b2, g2, be2,
                 W_out, b_out)
    return out.reshape(_BATCH, _SEQ, _VOCAB).astype(out_dtype)


# speedup vs baseline: 20.3450x; 1.2454x over previous
import functools

import jax
import jax.numpy as jnp
import numpy as np
from jax import lax
from jax.experimental import pallas as pl
from jax.experimental.pallas import tpu as pltpu
from jax.experimental.pallas import tpu_sc as plsc

_HASH_PRIMES = [2654435761, 2246822519, 3266489917, 2028178513, 1220703125, 1610612741, 805306457, 402653189, 3674653429, 2860486313, 1073676287, 2971215073, 1500450271, 3267000013, 2654435789, 4049292737, 2246822531, 3266489927, 2028178519, 1220703133, 1610612743, 805306459, 402653191, 3674653433, 2654435771, 2246822527, 3266489933, 2028178529, 1220703137, 1610612747, 805306463, 402653197, 3674653441, 2860486319, 1073676293, 2971215077, 1500450281, 3267000017, 2654435801, 4049292743, 2246822537, 3266489939, 2028178531, 1220703143, 1610612753, 805306467, 402653201, 3674653447]

_BATCH = 1024
_SEQ = 20
_N = _BATCH * _SEQ
_NUM_TABLES = 16
_BUCKETS = 100000
_BYTE_DIM = 128
_EMB = 16
_NF = _NUM_TABLES * _EMB
_IN_DIM = _BYTE_DIM + _NF
_HIDDEN = 512
_VOCAB = 1024


def _default_patterns(num_tables):
    patterns = []
    for offset in range(1, min(num_tables // 4 + 1, 9)):
        patterns.append((offset,))
    pairs = [(1, 2), (2, 3), (3, 4), (1, 3), (2, 4), (1, 4), (1, 5), (2, 5), (3, 5), (1, 6), (2, 6), (1, 7)]
    for p in pairs:
        if len(patterns) >= num_tables:
            break
        patterns.append(p)
    trigrams = [(1, 2, 3), (1, 2, 4), (1, 3, 5), (2, 3, 4)]
    for t in trigrams:
        if len(patterns) >= num_tables:
            break
        patterns.append(t)
    offset = 8
    while len(patterns) < num_tables:
        patterns.append((1, offset))
        offset += 1
    return tuple(patterns[:num_tables])


_PATTERNS = _default_patterns(_NUM_TABLES)
_MAX_OFF = max(max(p) for p in _PATTERNS)
_NB = _N // 128


def _mod_1e5(u):
    q = (u.astype(jnp.float32) * jnp.float32(1e-5)).astype(jnp.int32)
    r = u - q * 100000
    r = jnp.where(r < 0, r + 100000, r)
    r = jnp.where(r < 0, r + 100000, r)
    r = jnp.where(r >= 100000, r - 100000, r)
    r = jnp.where(r >= 100000, r - 100000, r)
    return r


def _hash_body(shift_ref, out_ref):
    shifts = [shift_ref[o] for o in range(_MAX_OFF)]
    c16 = jnp.int32(16)
    for t, pattern in enumerate(_PATTERNS):
        h_lo = jnp.zeros_like(shifts[0])
        h_hi = jnp.zeros_like(shifts[0])
        for k, off in enumerate(pattern):
            p = _HASH_PRIMES[(t * 3 + k) % len(_HASH_PRIMES)]
            p_hi, p_lo = p >> 16, p & 0xFFFF
            tok = shifts[off - 1]
            a = tok * p_hi
            b = tok * p_lo
            c = a + lax.shift_right_logical(b, c16)
            hi = lax.shift_right_logical(c, c16)
            lo = lax.shift_left(c & 0xFFFF, c16) | (b & 0xFFFF)
            h_hi = h_hi ^ hi
            h_lo = h_lo ^ lo
        s = lax.shift_right_logical(h_lo, jnp.int32(31))
        u = h_lo & 0x7FFFFFFF
        v = h_hi * 67296 + s * 83648 + _mod_1e5(u)
        idx = _mod_1e5(v)
        for e in range(_EMB):
            out_ref[t * _EMB + e] = idx + (t * _EMB + e) * _BUCKETS


def _compute_indices(shift_stack):
    return pl.pallas_call(
        _hash_body,
        out_shape=jax.ShapeDtypeStruct((_NF, _NB, 128), jnp.int32),
    )(shift_stack)


_NC = 2
_NS = 16
_NW = _NC * _NS
_C = 128
_SUB = _N // _NW // _C


def _sc_gather_body(tok_hbm, eidx_hbm, byte_hbm, tabs_hbm, byte_out, xt_out,
                    tok_v, eidx_v, byte_v, xt_v, sem):
    c = lax.axis_index("c")
    s = lax.axis_index("s")
    wid = s * jnp.int32(_NC) + c

    def body(j, carry):
        bb = wid * jnp.int32(_SUB) + j
        base = bb * jnp.int32(_C)
        pltpu.sync_copy(tok_hbm.at[pl.ds(base, _C)], tok_v)
        pltpu.sync_copy(eidx_hbm.at[:, bb], eidx_v)
        cp_byte = pltpu.async_copy(byte_hbm.at[tok_v], byte_v, sem)

        def inner(kk, cc):
            k0 = kk * jnp.int32(16)
            for u in range(16):
                k = k0 + jnp.int32(u)
                pltpu.async_copy(tabs_hbm.at[eidx_v.at[k]], xt_v.at[k], sem)
            pltpu.make_async_copy(
                xt_out.at[pl.ds(k0, 16), pl.ds(base, _C)],
                xt_v.at[pl.ds(k0, 16)], sem).wait()
            return cc

        lax.fori_loop(jnp.int32(0), jnp.int32(_NF // 16), inner, jnp.int32(0))
        cp_byte.wait()
        pltpu.sync_copy(byte_v, byte_out.at[pl.ds(base, _C)])
        pltpu.sync_copy(xt_v, xt_out.at[:, pl.ds(base, _C)])
        return carry

    lax.fori_loop(jnp.int32(0), jnp.int32(_SUB), body, jnp.int32(0))


def _sc_gather(tok_flat, eidx, byte_table, tabs_flat):
    mesh = plsc.VectorSubcoreMesh(core_axis_name="c", subcore_axis_name="s")
    f = functools.partial(
        pl.kernel,
        mesh=mesh,
        out_type=(
            jax.ShapeDtypeStruct((_N, _BYTE_DIM), jnp.float32),
            jax.ShapeDtypeStruct((_NF, _N), jnp.float32),
        ),
        scratch_types=[
            pltpu.VMEM((_C,), jnp.int32),
            pltpu.VMEM((_NF, _C), jnp.int32),
            pltpu.VMEM((_C, _BYTE_DIM), jnp.float32),
            pltpu.VMEM((_NF, _C), jnp.float32),
            pltpu.SemaphoreType.DMA,
        ],
    )(_sc_gather_body)
    return f(tok_flat, eidx, byte_table, tabs_flat)


_BN = 1024


def _dense_body(byte_ref, xt_ref, Wb_ref, Wt_ref, bin_ref, W1_ref, b1_ref,
                g1_ref, be1_ref, W2_ref, b2_ref, g2_ref, be2_ref,
                Wout_ref, bout_ref, o_ref):
    x = jnp.dot(byte_ref[...], Wb_ref[...], preferred_element_type=jnp.float32)
    x = x + lax.dot_general(xt_ref[...], Wt_ref[...],
                            (((0,), (0,)), ((), ())),
                            preferred_element_type=jnp.float32)
    x = x + bin_ref[...]
    for W_ref, b_ref, g_ref, be_ref in ((W1_ref, b1_ref, g1_ref, be1_ref),
                                        (W2_ref, b2_ref, g2_ref, be2_ref)):
        h = jnp.maximum(jnp.dot(x, W_ref[...], preferred_element_type=jnp.float32) + b_ref[...], 0.0)
        r = h + x
        mu = jnp.mean(r, axis=-1, keepdims=True)
        var = jnp.mean((r - mu) ** 2, axis=-1, keepdims=True)
        x = (r - mu) / jnp.sqrt(var + 1e-5) * g_ref[...] + be_ref[...]
    o_ref[...] = jnp.dot(x, Wout_ref[...], preferred_element_type=jnp.float32) + bout_ref[...]


def _dense(byte_e, xt, W_in, b_in, W1, b1, g1, be1, W2, b2, g2, be2, W_out, b_out):
    grid = (_N // _BN,)
    _z = np.int32(0)
    full = lambda shape: pl.BlockSpec(shape, lambda i: (_z, _z))
    return pl.pallas_call(
        _dense_body,
        grid=grid,
        in_specs=[
            pl.BlockSpec((_BN, _BYTE_DIM), lambda i: (i, _z)),
            pl.BlockSpec((_NF, _BN), lambda i: (_z, i)),
            full((_BYTE_DIM, _HIDDEN)), full((_NF, _HIDDEN)), full((1, _HIDDEN)),
            full((_HIDDEN, _HIDDEN)), full((1, _HIDDEN)), full((1, _HIDDEN)), full((1, _HIDDEN)),
            full((_HIDDEN, _HIDDEN)), full((1, _HIDDEN)), full((1, _HIDDEN)), full((1, _HIDDEN)),
            full((_HIDDEN, _VOCAB)), full((1, _VOCAB)),
        ],
        out_specs=pl.BlockSpec((_BN, _VOCAB), lambda i: (i, _z)),
        out_shape=jax.ShapeDtypeStruct((_N, _VOCAB), jnp.float32),
    )(byte_e, xt, W_in[:_BYTE_DIM], W_in[_BYTE_DIM:], b_in.reshape(1, -1),
      W1, b1.reshape(1, -1), g1.reshape(1, -1), be1.reshape(1, -1),
      W2, b2.reshape(1, -1), g2.reshape(1, -1), be2.reshape(1, -1),
      W_out, b_out.reshape(1, -1))


def kernel(tokens, byte_table, hash_tables, W_in, b_in, W1, b1, g1, be1,
           W2, b2, g2, be2, W_out, b_out):
    out_dtype = jnp.result_type(byte_table.dtype, W_in.dtype, W_out.dtype)
    f32 = jnp.float32
    byte_table = byte_table.astype(f32)
    hash_tables = hash_tables.astype(f32)
    W_in, b_in, W1, b1, g1, be1 = (a.astype(f32) for a in (W_in, b_in, W1, b1, g1, be1))
    W2, b2, g2, be2, W_out, b_out = (a.astype(f32) for a in (W2, b2, g2, be2, W_out, b_out))
    tokT = tokens.astype(jnp.int32).T
    shifts = [jnp.pad(tokT[:-o], ((o, 0), (0, 0))) for o in range(1, _MAX_OFF + 1)]
    shift_stack = jnp.stack(shifts).reshape(_MAX_OFF, _NB, 128)
    eidx = _compute_indices(shift_stack)
    tok_flat = tokT.reshape(_N)
    tabs_flat = jnp.transpose(hash_tables, (0, 2, 1)).reshape(
        _NUM_TABLES * _EMB * _BUCKETS)
    byte_e, xt = _sc_gather(tok_flat, eidx, byte_table, tabs_flat)
    out = _dense(byte_e, xt, W_in, b_in, W1, b1, g1, be1, W2, b2, g2, be2,
                 W_out, b_out)
    out = jnp.transpose(out.reshape(_SEQ, _BATCH, _VOCAB), (1, 0, 2))
    return out.astype(out_dtype)


# speedup vs baseline: 20.5274x; 1.0090x over previous
import functools

import jax
import jax.numpy as jnp
import numpy as np
from jax import lax
from jax.experimental import pallas as pl
from jax.experimental.pallas import tpu as pltpu
from jax.experimental.pallas import tpu_sc as plsc

_HASH_PRIMES = [2654435761, 2246822519, 3266489917, 2028178513, 1220703125, 1610612741, 805306457, 402653189, 3674653429, 2860486313, 1073676287, 2971215073, 1500450271, 3267000013, 2654435789, 4049292737, 2246822531, 3266489927, 2028178519, 1220703133, 1610612743, 805306459, 402653191, 3674653433, 2654435771, 2246822527, 3266489933, 2028178529, 1220703137, 1610612747, 805306463, 402653197, 3674653441, 2860486319, 1073676293, 2971215077, 1500450281, 3267000017, 2654435801, 4049292743, 2246822537, 3266489939, 2028178531, 1220703143, 1610612753, 805306467, 402653201, 3674653447]

_BATCH = 1024
_SEQ = 20
_N = _BATCH * _SEQ
_NUM_TABLES = 16
_BUCKETS = 100000
_BYTE_DIM = 128
_EMB = 16
_NF = _NUM_TABLES * _EMB
_IN_DIM = _BYTE_DIM + _NF
_HIDDEN = 512
_VOCAB = 1024


def _default_patterns(num_tables):
    patterns = []
    for offset in range(1, min(num_tables // 4 + 1, 9)):
        patterns.append((offset,))
    pairs = [(1, 2), (2, 3), (3, 4), (1, 3), (2, 4), (1, 4), (1, 5), (2, 5), (3, 5), (1, 6), (2, 6), (1, 7)]
    for p in pairs:
        if len(patterns) >= num_tables:
            break
        patterns.append(p)
    trigrams = [(1, 2, 3), (1, 2, 4), (1, 3, 5), (2, 3, 4)]
    for t in trigrams:
        if len(patterns) >= num_tables:
            break
        patterns.append(t)
    offset = 8
    while len(patterns) < num_tables:
        patterns.append((1, offset))
        offset += 1
    return tuple(patterns[:num_tables])


_PATTERNS = _default_patterns(_NUM_TABLES)
_MAX_OFF = max(max(p) for p in _PATTERNS)
_NB = _N // 128


def _mod_1e5(u):
    q = (u.astype(jnp.float32) * jnp.float32(1e-5)).astype(jnp.int32)
    r = u - q * 100000
    r = jnp.where(r < 0, r + 100000, r)
    r = jnp.where(r < 0, r + 100000, r)
    r = jnp.where(r >= 100000, r - 100000, r)
    r = jnp.where(r >= 100000, r - 100000, r)
    return r


def _hash_body(shift_ref, out_ref):
    shifts = [shift_ref[o] for o in range(_MAX_OFF)]
    c16 = jnp.int32(16)
    for t, pattern in enumerate(_PATTERNS):
        h_lo = jnp.zeros_like(shifts[0])
        h_hi = jnp.zeros_like(shifts[0])
        for k, off in enumerate(pattern):
            p = _HASH_PRIMES[(t * 3 + k) % len(_HASH_PRIMES)]
            p_hi, p_lo = p >> 16, p & 0xFFFF
            tok = shifts[off - 1]
            a = tok * p_hi
            b = tok * p_lo
            c = a + lax.shift_right_logical(b, c16)
            hi = lax.shift_right_logical(c, c16)
            lo = lax.shift_left(c & 0xFFFF, c16) | (b & 0xFFFF)
            h_hi = h_hi ^ hi
            h_lo = h_lo ^ lo
        s = lax.shift_right_logical(h_lo, jnp.int32(31))
        u = h_lo & 0x7FFFFFFF
        v = h_hi * 67296 + s * 83648 + _mod_1e5(u)
        out_ref[t] = _mod_1e5(v)


def _compute_indices(shift_stack):
    return pl.pallas_call(
        _hash_body,
        out_shape=jax.ShapeDtypeStruct((_NUM_TABLES, _NB, 128), jnp.int32),
    )(shift_stack)


_NC = 2
_NS = 16
_NW = _NC * _NS
_C = 128
_SUB = _N // _NW // _C


def _sc_gather_body(tok_hbm, gidx_hbm, byte_hbm, tabs_hbm, byte_out, xt_out,
                    tok_v, idx_v, byte_v, xt_v, sem):
    c = lax.axis_index("c")
    s = lax.axis_index("s")
    wid = s * jnp.int32(_NC) + c

    def body(j, carry):
        bb = wid * jnp.int32(_SUB) + j
        base = bb * jnp.int32(_C)
        pltpu.sync_copy(tok_hbm.at[pl.ds(base, _C)], tok_v)
        pltpu.sync_copy(gidx_hbm.at[:, bb], idx_v)
        cp_byte = pltpu.async_copy(byte_hbm.at[tok_v], byte_v, sem)

        def inner(kk, cc):
            k0 = kk * jnp.int32(16)
            for u in range(16):
                k = k0 + jnp.int32(u)
                off = k * jnp.int32(_BUCKETS)
                pltpu.async_copy(
                    tabs_hbm.at[pl.ds(off, _BUCKETS)].at[idx_v.at[kk]],
                    xt_v.at[k], sem)
            pltpu.make_async_copy(
                xt_out.at[pl.ds(k0, 16), pl.ds(base, _C)],
                xt_v.at[pl.ds(k0, 16)], sem).wait()
            return cc

        lax.fori_loop(jnp.int32(0), jnp.int32(_NUM_TABLES), inner, jnp.int32(0))
        cp_byte.wait()
        pltpu.sync_copy(byte_v, byte_out.at[pl.ds(base, _C)])
        pltpu.sync_copy(xt_v, xt_out.at[:, pl.ds(base, _C)])
        return carry

    lax.fori_loop(jnp.int32(0), jnp.int32(_SUB), body, jnp.int32(0))


def _sc_gather(tok_flat, gidx, byte_table, tabs_flat):
    mesh = plsc.VectorSubcoreMesh(core_axis_name="c", subcore_axis_name="s")
    f = functools.partial(
        pl.kernel,
        mesh=mesh,
        out_type=(
            jax.ShapeDtypeStruct((_N, _BYTE_DIM), jnp.float32),
            jax.ShapeDtypeStruct((_NF, _N), jnp.float32),
        ),
        scratch_types=[
            pltpu.VMEM((_C,), jnp.int32),
            pltpu.VMEM((_NUM_TABLES, _C), jnp.int32),
            pltpu.VMEM((_C, _BYTE_DIM), jnp.float32),
            pltpu.VMEM((_NF, _C), jnp.float32),
            pltpu.SemaphoreType.DMA,
        ],
    )(_sc_gather_body)
    return f(tok_flat, gidx, byte_table, tabs_flat)


_BN = 1024


def _dense_body(byte_ref, xt_ref, Wb_ref, Wt_ref, bin_ref, W1_ref, b1_ref,
                g1_ref, be1_ref, W2_ref, b2_ref, g2_ref, be2_ref,
                Wout_ref, bout_ref, o_ref):
    x = jnp.dot(byte_ref[...], Wb_ref[...], preferred_element_type=jnp.float32)
    x = x + lax.dot_general(xt_ref[...], Wt_ref[...],
                            (((0,), (0,)), ((), ())),
                            preferred_element_type=jnp.float32)
    x = x + bin_ref[...]
    for W_ref, b_ref, g_ref, be_ref in ((W1_ref, b1_ref, g1_ref, be1_ref),
                                        (W2_ref, b2_ref, g2_ref, be2_ref)):
        h = jnp.maximum(jnp.dot(x, W_ref[...], preferred_element_type=jnp.float32) + b_ref[...], 0.0)
        r = h + x
        mu = jnp.mean(r, axis=-1, keepdims=True)
        var = jnp.mean((r - mu) ** 2, axis=-1, keepdims=True)
        x = (r - mu) / jnp.sqrt(var + 1e-5) * g_ref[...] + be_ref[...]
    o_ref[...] = jnp.dot(x, Wout_ref[...], preferred_element_type=jnp.float32) + bout_ref[...]


def _dense(byte_e, xt, W_in, b_in, W1, b1, g1, be1, W2, b2, g2, be2, W_out, b_out):
    grid = (_N // _BN,)
    _z = np.int32(0)
    full = lambda shape: pl.BlockSpec(shape, lambda i: (_z, _z))
    return pl.pallas_call(
        _dense_body,
        grid=grid,
        in_specs=[
            pl.BlockSpec((_BN, _BYTE_DIM), lambda i: (i, _z)),
            pl.BlockSpec((_NF, _BN), lambda i: (_z, i)),
            full((_BYTE_DIM, _HIDDEN)), full((_NF, _HIDDEN)), full((1, _HIDDEN)),
            full((_HIDDEN, _HIDDEN)), full((1, _HIDDEN)), full((1, _HIDDEN)), full((1, _HIDDEN)),
            full((_HIDDEN, _HIDDEN)), full((1, _HIDDEN)), full((1, _HIDDEN)), full((1, _HIDDEN)),
            full((_HIDDEN, _VOCAB)), full((1, _VOCAB)),
        ],
        out_specs=pl.BlockSpec((_BN, _VOCAB), lambda i: (i, _z)),
        out_shape=jax.ShapeDtypeStruct((_N, _VOCAB), jnp.float32),
    )(byte_e, xt, W_in[:_BYTE_DIM], W_in[_BYTE_DIM:], b_in.reshape(1, -1),
      W1, b1.reshape(1, -1), g1.reshape(1, -1), be1.reshape(1, -1),
      W2, b2.reshape(1, -1), g2.reshape(1, -1), be2.reshape(1, -1),
      W_out, b_out.reshape(1, -1))


def kernel(tokens, byte_table, hash_tables, W_in, b_in, W1, b1, g1, be1,
           W2, b2, g2, be2, W_out, b_out):
    out_dtype = jnp.result_type(byte_table.dtype, W_in.dtype, W_out.dtype)
    f32 = jnp.float32
    byte_table = byte_table.astype(f32)
    hash_tables = hash_tables.astype(f32)
    W_in, b_in, W1, b1, g1, be1 = (a.astype(f32) for a in (W_in, b_in, W1, b1, g1, be1))
    W2, b2, g2, be2, W_out, b_out = (a.astype(f32) for a in (W2, b2, g2, be2, W_out, b_out))
    tokT = tokens.astype(jnp.int32).T
    shifts = [jnp.pad(tokT[:-o], ((o, 0), (0, 0))) for o in range(1, _MAX_OFF + 1)]
    shift_stack = jnp.stack(shifts).reshape(_MAX_OFF, _NB, 128)
    gidx = _compute_indices(shift_stack)
    tok_flat = tokT.reshape(_N)
    tabs_flat = jnp.transpose(hash_tables, (0, 2, 1)).reshape(
        _NUM_TABLES * _EMB * _BUCKETS)
    byte_e, xt = _sc_gather(tok_flat, gidx, byte_table, tabs_flat)
    out = _dense(byte_e, xt, W_in, b_in, W1, b1, g1, be1, W2, b2, g2, be2,
                 W_out, b_out)
    out = jnp.transpose(out.reshape(_SEQ, _BATCH, _VOCAB), (1, 0, 2))
    return out.astype(out_dtype)
